# Initial kernel scaffold; baseline (speedup 1.0000x reference)
import jax
import jax.numpy as jnp
from jax.experimental import pallas as pl


def kernel(x, edge_index, W1, b1, W2, b2):
    raise NotImplementedError("write your pallas kernel here")


# speedup vs baseline: 13.2262x; 13.2262x over previous
import functools

import jax
import jax.numpy as jnp
from jax import lax
from jax.experimental import pallas as pl
from jax.experimental.pallas import tpu as pltpu
from jax.experimental.pallas import tpu_sc as plsc

N = 10000
E = 320000
D = 128

NC = 2
NS = 16
NW = NC * NS
EW = E // NW
C = 80
M = EW // C

NP = 10240
RT = NP // NS

R = 1000


def _sc_mesh():
    return plsc.VectorSubcoreMesh(core_axis_name="c", subcore_axis_name="s")


def _sc_degree(dst, zeros_np):
    @functools.partial(
        pl.kernel,
        mesh=_sc_mesh(),
        out_type=jax.ShapeDtypeStruct((NC, NP), jnp.float32),
        scratch_types=[
            pltpu.VMEM((C,), jnp.int32),
            pltpu.VMEM((C,), jnp.float32),
            pltpu.VMEM_SHARED((NP,), jnp.float32),
        ],
    )
    def deg_kernel(dst_hbm, z_hbm, out_hbm, didx, ones_v, deg_sh):
        cid = lax.axis_index("c")
        sid = lax.axis_index("s")
        wid = sid * NC + cid
        for i in range(C // 16):
            ones_v[pl.ds(i * 16, 16)] = jnp.full((16,), 1.0, dtype=jnp.float32)
        pltpu.sync_copy(z_hbm.at[pl.ds(sid * RT, RT)],
                        deg_sh.at[pl.ds(sid * RT, RT)])
        plsc.subcore_barrier()

        def body(ci, carry):
            base = wid * EW + ci * C
            pltpu.sync_copy(dst_hbm.at[pl.ds(base, C)], didx)
            pltpu.sync_copy(ones_v, deg_sh.at[didx], add=True)
            return carry

        lax.fori_loop(0, M, body, 0)
        plsc.subcore_barrier()
        pltpu.sync_copy(deg_sh.at[pl.ds(sid * RT, RT)],
                        out_hbm.at[cid, pl.ds(sid * RT, RT)])

    return deg_kernel(dst, zeros_np)


def _sc_gather_scatter(g, src, dst, zeros_npd):
    @functools.partial(
        pl.kernel,
        mesh=_sc_mesh(),
        out_type=jax.ShapeDtypeStruct((NC, NP, D), jnp.float32),
        scratch_types=[
            pltpu.VMEM((C,), jnp.int32),
            pltpu.VMEM((C,), jnp.int32),
            pltpu.VMEM((C, D), jnp.float32),
            pltpu.VMEM_SHARED((NP, D), jnp.float32),
            pltpu.SemaphoreType.DMA,
        ],
    )
    def gs_kernel(g_hbm, src_hbm, dst_hbm, z_hbm, out_hbm,
                  sidx, didx, rows, acc_sh, sem):
        cid = lax.axis_index("c")
        sid = lax.axis_index("s")
        wid = sid * NC + cid
        pltpu.sync_copy(z_hbm.at[pl.ds(sid * RT, RT)],
                        acc_sh.at[pl.ds(sid * RT, RT)])
        plsc.subcore_barrier()

        def body(ci, carry):
            base = wid * EW + ci * C
            pltpu.sync_copy(src_hbm.at[pl.ds(base, C)], sidx)
            pltpu.sync_copy(dst_hbm.at[pl.ds(base, C)], didx)
            pltpu.async_copy(g_hbm.at[sidx], rows, sem).wait()
            pltpu.sync_copy(rows, acc_sh.at[didx], add=True)
            return carry

        lax.fori_loop(0, M, body, 0)
        plsc.subcore_barrier()
        pltpu.sync_copy(acc_sh.at[pl.ds(sid * RT, RT)],
                        out_hbm.at[cid, pl.ds(sid * RT, RT)])

    return gs_kernel(g, src, dst, zeros_npd)


def _tc_prescale(x, W, deg_col):
    def body(x_ref, w_ref, deg_ref, o_ref):
        dis = lax.rsqrt(deg_ref[...])
        h = jnp.dot(x_ref[...], w_ref[...], preferred_element_type=jnp.float32)
        o_ref[...] = h * dis

    return pl.pallas_call(
        body,
        grid=(N // R,),
        in_specs=[
            pl.BlockSpec((R, D), lambda i: (i, 0)),
            pl.BlockSpec((D, D), lambda i: (0, 0)),
            pl.BlockSpec((R, 1), lambda i: (i, 0)),
        ],
        out_specs=pl.BlockSpec((R, D), lambda i: (i, 0)),
        out_shape=jax.ShapeDtypeStruct((N, D), jnp.float32),
    )(x, W, deg_col)


def _tc_combine_layer2(acc0, acc1, g1, deg_col, b1_row, W2):
    def body(a0_ref, a1_ref, g_ref, deg_ref, b_ref, w_ref, o_ref):
        dis = lax.rsqrt(deg_ref[...])
        z = (a0_ref[...] + a1_ref[...] + g_ref[...]) * dis + b_ref[...]
        z = jnp.maximum(z, 0.0)
        o_ref[...] = jnp.dot(z, w_ref[...],
                             preferred_element_type=jnp.float32) * dis

    return pl.pallas_call(
        body,
        grid=(N // R,),
        in_specs=[
            pl.BlockSpec((R, D), lambda i: (i, 0)),
            pl.BlockSpec((R, D), lambda i: (i, 0)),
            pl.BlockSpec((R, D), lambda i: (i, 0)),
            pl.BlockSpec((R, 1), lambda i: (i, 0)),
            pl.BlockSpec((1, D), lambda i: (0, 0)),
            pl.BlockSpec((D, D), lambda i: (0, 0)),
        ],
        out_specs=pl.BlockSpec((R, D), lambda i: (i, 0)),
        out_shape=jax.ShapeDtypeStruct((N, D), jnp.float32),
    )(acc0, acc1, g1, deg_col, b1_row, W2)


def _tc_combine_out(acc0, acc1, g2, deg_col, b2_row):
    def body(a0_ref, a1_ref, g_ref, deg_ref, b_ref, o_ref):
        dis = lax.rsqrt(deg_ref[...])
        o_ref[...] = (a0_ref[...] + a1_ref[...] + g_ref[...]) * dis + b_ref[...]

    return pl.pallas_call(
        body,
        grid=(N // R,),
        in_specs=[
            pl.BlockSpec((R, D), lambda i: (i, 0)),
            pl.BlockSpec((R, D), lambda i: (i, 0)),
            pl.BlockSpec((R, D), lambda i: (i, 0)),
            pl.BlockSpec((R, 1), lambda i: (i, 0)),
            pl.BlockSpec((1, D), lambda i: (0, 0)),
        ],
        out_specs=pl.BlockSpec((R, D), lambda i: (i, 0)),
        out_shape=jax.ShapeDtypeStruct((N, D), jnp.float32),
    )(acc0, acc1, g2, deg_col, b2_row)


def kernel(x, edge_index, W1, b1, W2, b2):
    src = edge_index[0].astype(jnp.int32)
    dst = edge_index[1].astype(jnp.int32)
    zeros_np = jnp.zeros((NP,), jnp.float32)
    zeros_npd = jnp.zeros((NP, D), jnp.float32)
    b1_row = b1.reshape(1, D)
    b2_row = b2.reshape(1, D)

    degp = _sc_degree(dst, zeros_np)
    deg_col = (degp[0, :N] + degp[1, :N] + 1.0).reshape(N, 1)

    g1 = _tc_prescale(x, W1, deg_col)
    acc1 = _sc_gather_scatter(g1, src, dst, zeros_npd)
    g2 = _tc_combine_layer2(acc1[0, :N], acc1[1, :N], g1,
                            deg_col, b1_row, W2)
    acc2 = _sc_gather_scatter(g2, src, dst, zeros_npd)
    return _tc_combine_out(acc2[0, :N], acc2[1, :N], g2,
                           deg_col, b2_row)
